# baseline (speedup 1.0000x reference)
import functools
import math

import jax
import jax.numpy as jnp
from jax import lax
from jax.experimental import pallas as pl
from jax.experimental.pallas import tpu as pltpu
from jax.experimental.pallas import tpu_sc as plsc

_EMBED_DIM = 64
_SCALE = math.sqrt(_EMBED_DIM)
_CH = 128


@functools.lru_cache(maxsize=None)
def _build(B, D, vocab):
    info = plsc.get_sparse_core_info()
    NC, NS = info.num_cores, info.num_subcores
    NW = NC * NS
    n_ch = B // (NW * _CH)
    assert B == NW * n_ch * _CH

    mesh = plsc.VectorSubcoreMesh(core_axis_name="c", subcore_axis_name="s")

    @functools.partial(
        pl.kernel,
        mesh=mesh,
        out_type=jax.ShapeDtypeStruct((B, D), jnp.float32),
        scratch_types=[
            pltpu.VMEM((n_ch, _CH), jnp.int32),
            pltpu.VMEM((2, _CH, D), jnp.float32),
            pltpu.SemaphoreType.DMA,
        ],
        compiler_params=pltpu.CompilerParams(use_tc_tiling_on_sc=False),
    )
    def emb(x_hbm, lut_hbm, out_hbm, idx_v, rows_v, gsem):
        wid = lax.axis_index("s") * NC + lax.axis_index("c")
        base = wid * n_ch
        pltpu.sync_copy(x_hbm.at[pl.ds(base, n_ch)], idx_v)

        def chunk2(jj, carry):
            for b in range(2):
                j = jj * 2 + b
                pltpu.async_copy(
                    lut_hbm.at[idx_v.at[j]], rows_v.at[b], gsem
                ).wait()

                def mul_row(r, c2):
                    for c in range(D // 16):
                        sl = (b, r, pl.ds(c * 16, 16))
                        rows_v[sl] = rows_v[sl] * _SCALE
                    return c2

                lax.fori_loop(0, _CH, mul_row, 0)
                pltpu.sync_copy(
                    rows_v.at[b], out_hbm.at[pl.ds((base + j) * _CH, _CH)]
                )
            return carry

        lax.fori_loop(0, n_ch // 2, chunk2, 0)

    return emb


def kernel(x, lut):
    B0, S = x.shape
    D = lut.shape[1]
    idx = x.reshape(-1).astype(jnp.int32)
    B = idx.shape[0]
    info = plsc.get_sparse_core_info()
    NW = info.num_cores * info.num_subcores
    idx2 = idx.reshape(NW * (B // (NW * _CH)), _CH)
    out = _build(B, D, lut.shape[0])(idx2, lut)
    return out.reshape(B0, S, D)
